# Initial kernel scaffold; baseline (speedup 1.0000x reference)
import jax
import jax.numpy as jnp
from jax.experimental import pallas as pl


def kernel(x, embedding):
    raise NotImplementedError("write your pallas kernel here")


# speedup vs baseline: 1.2783x; 1.2783x over previous
import functools
import math

import jax
import jax.numpy as jnp
from jax import lax
from jax.experimental import pallas as pl
from jax.experimental.pallas import tpu as pltpu
from jax.experimental.pallas import tpu_sc as plsc

D_MODEL = 768
_SCALE = math.sqrt(D_MODEL)
_LANES = 16


def _emb_lookup_sc(x_flat, embedding, chunk_rows):
    B = x_flat.shape[0]
    info = plsc.get_sparse_core_info()
    nc, ns = info.num_cores, info.num_subcores
    nw = nc * ns
    b_per_w = B // nw
    nch = b_per_w // chunk_rows
    idx3 = x_flat.reshape(nw, nch, chunk_rows)
    mesh = plsc.VectorSubcoreMesh(core_axis_name="c", subcore_axis_name="s")

    @functools.partial(
        pl.kernel,
        mesh=mesh,
        out_type=jax.ShapeDtypeStruct((B, D_MODEL), jnp.float32),
        scratch_types=[
            pltpu.VMEM((nch, chunk_rows), jnp.int32),
            pltpu.VMEM((chunk_rows, D_MODEL), jnp.float32),
            pltpu.SemaphoreType.DMA,
        ],
    )
    def body(idx_hbm, table_hbm, out_hbm, idx_v, rows_v, sem):
        cid = lax.axis_index("c")
        sid = lax.axis_index("s")
        wid = sid * nc + cid
        base = wid * b_per_w
        pltpu.sync_copy(idx_hbm.at[wid], idx_v)

        def chunk(g, carry):
            pltpu.async_copy(table_hbm.at[idx_v.at[g]], rows_v, sem).wait()

            def row(r, c2):
                for c in range(D_MODEL // _LANES):
                    sl = pl.ds(c * _LANES, _LANES)
                    rows_v[r, sl] = rows_v[r, sl] * _SCALE
                return c2

            lax.fori_loop(0, chunk_rows, row, 0)
            pltpu.sync_copy(rows_v, out_hbm.at[pl.ds(base + g * chunk_rows, chunk_rows)])
            return carry

        lax.fori_loop(0, nch, chunk, 0)

    return body(idx3, embedding)


def kernel(x, embedding):
    b0, b1 = x.shape
    x_flat = x.reshape(b0 * b1).astype(jnp.int32)
    out = _emb_lookup_sc(x_flat, embedding, chunk_rows=128)
    return out.reshape(b0, b1, D_MODEL)


# speedup vs baseline: 1.6622x; 1.3003x over previous
import functools
import math

import jax
import jax.numpy as jnp
from jax import lax
from jax.experimental import pallas as pl
from jax.experimental.pallas import tpu as pltpu
from jax.experimental.pallas import tpu_sc as plsc

D_MODEL = 768
_SCALE = math.sqrt(D_MODEL)
_LANES = 16
_NBUF = 4


def _emb_lookup_sc(x_flat, embedding, chunk_rows):
    B = x_flat.shape[0]
    info = plsc.get_sparse_core_info()
    nc, ns = info.num_cores, info.num_subcores
    nw = nc * ns
    b_per_w = B // nw
    nch = b_per_w // chunk_rows
    assert nch % _NBUF == 0 and nch >= 2 * _NBUF
    idx3 = x_flat.reshape(nw, nch, chunk_rows)
    mesh = plsc.VectorSubcoreMesh(core_axis_name="c", subcore_axis_name="s")

    @functools.partial(
        pl.kernel,
        mesh=mesh,
        out_type=jax.ShapeDtypeStruct((B, D_MODEL), jnp.float32),
        scratch_types=[
            pltpu.VMEM((nch, chunk_rows), jnp.int32),
            pltpu.VMEM((_NBUF, chunk_rows, D_MODEL), jnp.float32),
            [pltpu.SemaphoreType.DMA] * _NBUF,
            [pltpu.SemaphoreType.DMA] * _NBUF,
        ],
    )
    def body(idx_hbm, table_hbm, out_hbm, idx_v, rows_v, gsems, ssems):
        cid = lax.axis_index("c")
        sid = lax.axis_index("s")
        wid = sid * nc + cid
        base = wid * b_per_w
        pltpu.sync_copy(idx_hbm.at[wid], idx_v)

        def gather(g, b):
            return pltpu.make_async_copy(
                table_hbm.at[idx_v.at[g]], rows_v.at[b], gsems[b]
            )

        def scatter(g, b):
            return pltpu.make_async_copy(
                rows_v.at[b],
                out_hbm.at[pl.ds(base + g * chunk_rows, chunk_rows)],
                ssems[b],
            )

        gather(0, 0).start()
        gather(1, 1).start()

        def step(i, carry):
            for b in range(_NBUF):
                g = i * _NBUF + b
                bn = (b + 2) % _NBUF

                @pl.when(g >= 2)
                def _():
                    scatter(g - 2, bn).wait()

                @pl.when(g + 2 < nch)
                def _():
                    gather(g + 2, bn).start()

                gather(g, b).wait()

                def row(r, c2):
                    for c in range(D_MODEL // _LANES):
                        sl = pl.ds(c * _LANES, _LANES)
                        rows_v[b, r, sl] = rows_v[b, r, sl] * _SCALE
                    return c2

                lax.fori_loop(0, chunk_rows, row, 0)
                scatter(g, b).start()
            return carry

        lax.fori_loop(0, nch // _NBUF, step, 0)
        scatter(nch - 2, (nch - 2) % _NBUF).wait()
        scatter(nch - 1, (nch - 1) % _NBUF).wait()

    return body(idx3, embedding)


def kernel(x, embedding):
    b0, b1 = x.shape
    x_flat = x.reshape(b0 * b1).astype(jnp.int32)
    out = _emb_lookup_sc(x_flat, embedding, chunk_rows=32)
    return out.reshape(b0, b1, D_MODEL)
